# baseline (speedup 1.0000x reference)
import functools

import jax
import jax.numpy as jnp
from jax import lax
from jax.experimental import pallas as pl
from jax.experimental.pallas import tpu as pltpu
from jax.experimental.pallas import tpu_sc as plsc

TOPK = 8
BN = 256
NW = 32
CT = 16
LANES = 16
ALPHA_P1 = 1.25


def _topk_body(z_ref, cb_ref, idx_ref, w_ref):
    z = z_ref[...]
    cb = cb_ref[...]
    prod = jax.lax.dot_general(z, cb, (((1,), (1,)), ((), ())),
                               preferred_element_type=jnp.float32)
    e2 = jnp.sum(cb * cb, axis=1)
    z2 = jnp.sum(z * z, axis=1, keepdims=True)
    s = (z2 + e2[None, :]) - 2.0 * prod

    E = s.shape[1]
    bn = s.shape[0]
    NL = 128
    NG = E // NL
    INF = jnp.float32(jnp.inf)
    SENT = jnp.int32(1 << 20)
    BIGK = jnp.int32(1 << 30)

    lane = jax.lax.broadcasted_iota(jnp.int32, (bn, NL), 1)
    M1 = jnp.full((bn, NL), INF)
    M2 = jnp.full((bn, NL), INF)
    M3 = jnp.full((bn, NL), INF)
    M4 = jnp.full((bn, NL), INF)
    G1 = jnp.full((bn, NL), SENT, jnp.int32)
    G2 = jnp.full((bn, NL), SENT, jnp.int32)
    G3 = jnp.full((bn, NL), SENT, jnp.int32)
    for c in range(NG):
        x = s[:, c * NL:(c + 1) * NL]
        cc = jnp.int32(c)
        b1 = x < M1
        d1v = jnp.where(b1, M1, x)
        d1g = jnp.where(b1, G1, cc)
        M1 = jnp.where(b1, x, M1)
        G1 = jnp.where(b1, cc, G1)
        b2 = d1v < M2
        d2v = jnp.where(b2, M2, d1v)
        d2g = jnp.where(b2, G2, d1g)
        M2 = jnp.where(b2, d1v, M2)
        G2 = jnp.where(b2, d1g, G2)
        b3 = d2v < M3
        d3v = jnp.where(b3, M3, d2v)
        M3 = jnp.where(b3, d2v, M3)
        G3 = jnp.where(b3, d2g, G3)
        M4 = jnp.minimum(M4, d3v)

    vals, idxs = [], []
    fb = jnp.zeros((bn, 1), jnp.int32)
    for _ in range(TOPK):
        v = jnp.min(M1, axis=1, keepdims=True)
        jkey = jnp.where(M1 == v, G1 * NL + lane, BIGK)
        j = jnp.min(jkey, axis=1, keepdims=True)
        fb = fb | (j >= SENT * NL).astype(jnp.int32)
        vals.append(v)
        idxs.append(j)
        wl = jkey == j
        M1 = jnp.where(wl, M2, M1)
        G1 = jnp.where(wl, G2, G1)
        M2 = jnp.where(wl, M3, M2)
        G2 = jnp.where(wl, G3, G2)
        M3 = jnp.where(wl, M4, M3)
        G3 = jnp.where(wl, SENT, G3)
        M4 = jnp.where(wl, INF, M4)
    v = jnp.concatenate(vals, axis=1)
    ind = jnp.concatenate(idxs, axis=1)

    w = jnp.exp(v[:, :1] - v)
    w = w / jnp.sum(w, axis=1, keepdims=True)

    idx_ref[...] = ind
    w_ref[...] = jnp.concatenate(
        [jnp.broadcast_to(w[:, k:k + 1], (w.shape[0], LANES))
         for k in range(TOPK)], axis=1)

    @pl.when(jnp.sum(fb) > 0)
    def _exact_fallback():
        ss = s
        col = jax.lax.broadcasted_iota(jnp.int32, ss.shape, 1)
        fvals, fidxs = [], []
        for _ in range(TOPK):
            m = jnp.min(ss, axis=1, keepdims=True)
            a = jnp.min(jnp.where(ss == m, col, E), axis=1, keepdims=True)
            fvals.append(m)
            fidxs.append(a)
            ss = jnp.where(col == a, INF, ss)
        fv = jnp.concatenate(fvals, axis=1)
        find = jnp.concatenate(fidxs, axis=1).astype(jnp.int32)
        fw = jnp.exp(fv[:, :1] - fv)
        fw = fw / jnp.sum(fw, axis=1, keepdims=True)
        idx_ref[...] = find
        w_ref[...] = jnp.concatenate(
            [jnp.broadcast_to(fw[:, k:k + 1], (fw.shape[0], LANES))
             for k in range(TOPK)], axis=1)


def _sc_combine_body(cb_hbm, z_hbm, idxf_hbm, wf_hbm, zq_hbm, loss_hbm,
                     idx_v0, w_v0, rows_v0, z_v0,
                     idx_v1, w_v1, rows_v1, z_v1,
                     out_v, lacc_v,
                     si0, sw0, sz0, sr0, si1, sw1, sz1, sr1):
    wid = lax.axis_index("s") * 2 + lax.axis_index("c")
    n = z_hbm.shape[0]
    d = z_hbm.shape[1]
    nc = d // LANES
    tpw = n // NW
    base_tok = wid * tpw
    nch = tpw // CT

    bufs = [(idx_v0, w_v0, rows_v0, z_v0, si0, sw0, sz0, sr0),
            (idx_v1, w_v1, rows_v1, z_v1, si1, sw1, sz1, sr1)]
    handles = {}

    def start_pre(g):
        idx_v, w_v, _, z_v, si, sw, sz, _ = bufs[g % 2]
        tok0 = base_tok + g * CT
        handles[(g, "i")] = pltpu.async_copy(
            idxf_hbm.at[pl.ds(tok0 * TOPK, CT * TOPK)], idx_v, si)
        handles[(g, "w")] = pltpu.async_copy(
            wf_hbm.at[pl.ds(tok0 * TOPK, CT * TOPK)], w_v, sw)
        handles[(g, "z")] = pltpu.async_copy(
            z_hbm.at[pl.ds(tok0, CT)], z_v, sz)

    def start_gather(g):
        idx_v, _, rows_v, _, _, _, _, sr = bufs[g % 2]
        handles[(g, "i")].wait()
        handles[(g, "r")] = pltpu.async_copy(cb_hbm.at[idx_v], rows_v, sr)

    start_pre(0)
    start_gather(0)
    if nch > 1:
        start_pre(1)

    lacc = jnp.zeros((LANES,), jnp.float32)
    for g in range(nch):
        _, w_v, rows_v, z_v, _, _, _, _ = bufs[g % 2]
        if g + 1 < nch:
            start_gather(g + 1)
        handles[(g, "w")].wait()
        handles[(g, "z")].wait()
        handles[(g, "r")].wait()

        def tok_body(t2, lacc, w_v=w_v, rows_v=rows_v, z_v=z_v):
            for dt in range(2):
                t = t2 * 2 + dt
                accs = [jnp.zeros((LANES,), jnp.float32) for _ in range(nc)]
                for k in range(TOPK):
                    wk = w_v[t * TOPK + k, :]
                    for c in range(nc):
                        accs[c] = accs[c] + wk * rows_v[t * TOPK + k,
                                                        pl.ds(c * LANES, LANES)]
                for c in range(nc):
                    out_v[t, pl.ds(c * LANES, LANES)] = accs[c]
                    dd = accs[c] - z_v[t, pl.ds(c * LANES, LANES)]
                    lacc = lacc + dd * dd
            return lacc

        lacc = lax.fori_loop(0, CT // 2, tok_body, lacc)
        tok0 = base_tok + g * CT
        pltpu.sync_copy(out_v, zq_hbm.at[pl.ds(tok0, CT)])
        if g + 2 < nch:
            start_pre(g + 2)

    lacc_v[...] = lacc
    pltpu.sync_copy(lacc_v, loss_hbm.at[wid])


NSPLIT = 2


def _run_tc(z_part, codebook):
    n, d = z_part.shape
    e = codebook.shape[0]
    return pl.pallas_call(
        _topk_body,
        grid=(n // BN,),
        in_specs=[
            pl.BlockSpec((BN, d), lambda i: (i, 0)),
            pl.BlockSpec((e, d), lambda i: (0, 0)),
        ],
        out_specs=[
            pl.BlockSpec((BN, TOPK), lambda i: (i, 0)),
            pl.BlockSpec((BN, TOPK * LANES), lambda i: (i, 0)),
        ],
        out_shape=[
            jax.ShapeDtypeStruct((n, TOPK), jnp.int32),
            jax.ShapeDtypeStruct((n, TOPK * LANES), jnp.float32),
        ],
    )(z_part, codebook)


def _run_sc(codebook, z_part, ind, w):
    n, d = z_part.shape
    idx_flat = ind.reshape(-1)
    w_flat = w.reshape(n * TOPK, LANES)
    mesh = plsc.VectorSubcoreMesh(core_axis_name="c", subcore_axis_name="s")
    sc = functools.partial(
        pl.kernel, mesh=mesh,
        out_type=[
            jax.ShapeDtypeStruct((n, d), jnp.float32),
            jax.ShapeDtypeStruct((NW, LANES), jnp.float32),
        ],
        scratch_types=(
            [pltpu.VMEM((CT * TOPK,), jnp.int32),
             pltpu.VMEM((CT * TOPK, LANES), jnp.float32),
             pltpu.VMEM((CT * TOPK, d), jnp.float32),
             pltpu.VMEM((CT, d), jnp.float32)] * 2
            + [pltpu.VMEM((CT, d), jnp.float32),
               pltpu.VMEM((LANES,), jnp.float32)]
            + [pltpu.SemaphoreType.DMA] * 8
        ),
    )(_sc_combine_body)
    return sc(codebook, z_part, idx_flat, w_flat)


def kernel(z, codebook):
    n, d = z.shape
    np_ = n // NSPLIT
    parts = [_run_tc(z[p * np_:(p + 1) * np_], codebook) for p in range(NSPLIT)]
    combs = [_run_sc(codebook, z[p * np_:(p + 1) * np_], ind, w)
             for p, (ind, w) in enumerate(parts)]

    ind = jnp.concatenate([p[0] for p in parts], axis=0)
    zq = jnp.concatenate([c[0] for c in combs], axis=0)
    q_loss = ALPHA_P1 * sum(jnp.sum(c[1]) for c in combs) / (n * d)
    zq_st = z + (zq - z)
    return (zq_st, ind, q_loss)


# speedup vs baseline: 1.0057x; 1.0057x over previous
import functools

import jax
import jax.numpy as jnp
from jax import lax
from jax.experimental import pallas as pl
from jax.experimental.pallas import tpu as pltpu
from jax.experimental.pallas import tpu_sc as plsc

TOPK = 8
BN = 256
NW = 32
CT = 16
LANES = 16
ALPHA_P1 = 1.25


def _topk_body(z_ref, cb_ref, idx_ref, w_ref):
    z = z_ref[...]
    cb = cb_ref[...]
    prod = jax.lax.dot_general(z, cb, (((1,), (1,)), ((), ())),
                               preferred_element_type=jnp.float32)
    e2 = jnp.sum(cb * cb, axis=1)
    z2 = jnp.sum(z * z, axis=1, keepdims=True)
    s = (z2 + e2[None, :]) - 2.0 * prod

    E = s.shape[1]
    bn = s.shape[0]
    NL = 128
    NG = E // NL
    INF = jnp.float32(jnp.inf)
    SENT = jnp.int32(1 << 20)
    BIGK = jnp.int32(1 << 30)

    lane = jax.lax.broadcasted_iota(jnp.int32, (bn, NL), 1)
    M1 = jnp.full((bn, NL), INF)
    M2 = jnp.full((bn, NL), INF)
    M3 = jnp.full((bn, NL), INF)
    M4 = jnp.full((bn, NL), INF)
    G1 = jnp.full((bn, NL), SENT, jnp.int32)
    G2 = jnp.full((bn, NL), SENT, jnp.int32)
    G3 = jnp.full((bn, NL), SENT, jnp.int32)
    for c in range(NG):
        x = s[:, c * NL:(c + 1) * NL]
        cc = jnp.int32(c)
        b1 = x < M1
        d1v = jnp.where(b1, M1, x)
        d1g = jnp.where(b1, G1, cc)
        M1 = jnp.where(b1, x, M1)
        G1 = jnp.where(b1, cc, G1)
        b2 = d1v < M2
        d2v = jnp.where(b2, M2, d1v)
        d2g = jnp.where(b2, G2, d1g)
        M2 = jnp.where(b2, d1v, M2)
        G2 = jnp.where(b2, d1g, G2)
        b3 = d2v < M3
        d3v = jnp.where(b3, M3, d2v)
        M3 = jnp.where(b3, d2v, M3)
        G3 = jnp.where(b3, d2g, G3)
        M4 = jnp.minimum(M4, d3v)

    vals, idxs = [], []
    fb = jnp.zeros((bn, 1), jnp.int32)
    for _ in range(TOPK):
        v = jnp.min(M1, axis=1, keepdims=True)
        jkey = jnp.where(M1 == v, G1 * NL + lane, BIGK)
        j = jnp.min(jkey, axis=1, keepdims=True)
        fb = fb | (j >= SENT * NL).astype(jnp.int32)
        vals.append(v)
        idxs.append(j)
        wl = jkey == j
        M1 = jnp.where(wl, M2, M1)
        G1 = jnp.where(wl, G2, G1)
        M2 = jnp.where(wl, M3, M2)
        G2 = jnp.where(wl, G3, G2)
        M3 = jnp.where(wl, M4, M3)
        G3 = jnp.where(wl, SENT, G3)
        M4 = jnp.where(wl, INF, M4)
    v = jnp.concatenate(vals, axis=1)
    ind = jnp.concatenate(idxs, axis=1)

    w = jnp.exp(v[:, :1] - v)
    w = w / jnp.sum(w, axis=1, keepdims=True)

    idx_ref[...] = ind
    w_ref[...] = jnp.concatenate(
        [jnp.broadcast_to(w[:, k:k + 1], (w.shape[0], LANES))
         for k in range(TOPK)], axis=1)

    @pl.when(jnp.sum(fb) > 0)
    def _exact_fallback():
        ss = s
        col = jax.lax.broadcasted_iota(jnp.int32, ss.shape, 1)
        fvals, fidxs = [], []
        for _ in range(TOPK):
            m = jnp.min(ss, axis=1, keepdims=True)
            a = jnp.min(jnp.where(ss == m, col, E), axis=1, keepdims=True)
            fvals.append(m)
            fidxs.append(a)
            ss = jnp.where(col == a, INF, ss)
        fv = jnp.concatenate(fvals, axis=1)
        find = jnp.concatenate(fidxs, axis=1).astype(jnp.int32)
        fw = jnp.exp(fv[:, :1] - fv)
        fw = fw / jnp.sum(fw, axis=1, keepdims=True)
        idx_ref[...] = find
        w_ref[...] = jnp.concatenate(
            [jnp.broadcast_to(fw[:, k:k + 1], (fw.shape[0], LANES))
             for k in range(TOPK)], axis=1)


def _sc_combine_body(cb_hbm, z_hbm, idxf_hbm, wf_hbm, zq_hbm, loss_hbm,
                     idx_v0, w_v0, rows_v0, z_v0,
                     idx_v1, w_v1, rows_v1, z_v1,
                     out_v, lacc_v,
                     si0, sw0, sz0, sr0, si1, sw1, sz1, sr1):
    wid = lax.axis_index("s") * 2 + lax.axis_index("c")
    n = z_hbm.shape[0]
    d = z_hbm.shape[1]
    nc = d // LANES
    tpw = n // NW
    base_tok = wid * tpw
    nch = tpw // CT

    bufs = [(idx_v0, w_v0, rows_v0, z_v0, si0, sw0, sz0, sr0),
            (idx_v1, w_v1, rows_v1, z_v1, si1, sw1, sz1, sr1)]
    handles = {}

    def start_pre(g):
        idx_v, w_v, _, z_v, si, sw, sz, _ = bufs[g % 2]
        tok0 = base_tok + g * CT
        handles[(g, "i")] = pltpu.async_copy(
            idxf_hbm.at[pl.ds(tok0 * TOPK, CT * TOPK)], idx_v, si)
        handles[(g, "w")] = pltpu.async_copy(
            wf_hbm.at[pl.ds(tok0 * TOPK, CT * TOPK)], w_v, sw)
        handles[(g, "z")] = pltpu.async_copy(
            z_hbm.at[pl.ds(tok0, CT)], z_v, sz)

    def start_gather(g):
        idx_v, _, rows_v, _, _, _, _, sr = bufs[g % 2]
        handles[(g, "i")].wait()
        handles[(g, "r")] = pltpu.async_copy(cb_hbm.at[idx_v], rows_v, sr)

    start_pre(0)
    start_gather(0)
    if nch > 1:
        start_pre(1)

    lacc = jnp.zeros((LANES,), jnp.float32)
    for g in range(nch):
        _, w_v, rows_v, z_v, _, _, _, _ = bufs[g % 2]
        if g + 1 < nch:
            start_gather(g + 1)
        handles[(g, "w")].wait()
        handles[(g, "z")].wait()
        handles[(g, "r")].wait()

        def tok_body(t, lacc, w_v=w_v, rows_v=rows_v, z_v=z_v):
            accs = [jnp.zeros((LANES,), jnp.float32) for _ in range(nc)]
            for k in range(TOPK):
                wk = w_v[t * TOPK + k, :]
                for c in range(nc):
                    accs[c] = accs[c] + wk * rows_v[t * TOPK + k,
                                                    pl.ds(c * LANES, LANES)]
            for c in range(nc):
                out_v[t, pl.ds(c * LANES, LANES)] = accs[c]
                dd = accs[c] - z_v[t, pl.ds(c * LANES, LANES)]
                lacc = lacc + dd * dd
            return lacc

        lacc = lax.fori_loop(0, CT, tok_body, lacc)
        tok0 = base_tok + g * CT
        pltpu.sync_copy(out_v, zq_hbm.at[pl.ds(tok0, CT)])
        if g + 2 < nch:
            start_pre(g + 2)

    lacc_v[...] = lacc
    pltpu.sync_copy(lacc_v, loss_hbm.at[wid])


NSPLIT = 2


def _run_tc(z_part, codebook):
    n, d = z_part.shape
    e = codebook.shape[0]
    return pl.pallas_call(
        _topk_body,
        grid=(n // BN,),
        in_specs=[
            pl.BlockSpec((BN, d), lambda i: (i, 0)),
            pl.BlockSpec((e, d), lambda i: (0, 0)),
        ],
        out_specs=[
            pl.BlockSpec((BN, TOPK), lambda i: (i, 0)),
            pl.BlockSpec((BN, TOPK * LANES), lambda i: (i, 0)),
        ],
        out_shape=[
            jax.ShapeDtypeStruct((n, TOPK), jnp.int32),
            jax.ShapeDtypeStruct((n, TOPK * LANES), jnp.float32),
        ],
    )(z_part, codebook)


def _run_sc(codebook, z_part, ind, w):
    n, d = z_part.shape
    idx_flat = ind.reshape(-1)
    w_flat = w.reshape(n * TOPK, LANES)
    mesh = plsc.VectorSubcoreMesh(core_axis_name="c", subcore_axis_name="s")
    sc = functools.partial(
        pl.kernel, mesh=mesh,
        out_type=[
            jax.ShapeDtypeStruct((n, d), jnp.float32),
            jax.ShapeDtypeStruct((NW, LANES), jnp.float32),
        ],
        scratch_types=(
            [pltpu.VMEM((CT * TOPK,), jnp.int32),
             pltpu.VMEM((CT * TOPK, LANES), jnp.float32),
             pltpu.VMEM((CT * TOPK, d), jnp.float32),
             pltpu.VMEM((CT, d), jnp.float32)] * 2
            + [pltpu.VMEM((CT, d), jnp.float32),
               pltpu.VMEM((LANES,), jnp.float32)]
            + [pltpu.SemaphoreType.DMA] * 8
        ),
    )(_sc_combine_body)
    return sc(codebook, z_part, idx_flat, w_flat)


def kernel(z, codebook):
    n, d = z.shape
    np_ = n // NSPLIT
    parts = [_run_tc(z[p * np_:(p + 1) * np_], codebook) for p in range(NSPLIT)]
    combs = [_run_sc(codebook, z[p * np_:(p + 1) * np_], ind, w)
             for p, (ind, w) in enumerate(parts)]

    ind = jnp.concatenate([p[0] for p in parts], axis=0)
    zq = jnp.concatenate([c[0] for c in combs], axis=0)
    q_loss = ALPHA_P1 * sum(jnp.sum(c[1]) for c in combs) / (n * d)
    zq_st = z + (zq - z)
    return (zq_st, ind, q_loss)
